# baseline (speedup 1.0000x reference)
import functools

import jax
import jax.numpy as jnp
from jax import lax
from jax.experimental import pallas as pl
from jax.experimental.pallas import tpu as pltpu
from jax.experimental.pallas import tpu_sc as plsc

NUM_FIELDS = 26
FIELD_SIZE = 100000
ROWS = 16384
DIM = 16
B_TOTAL = ROWS * NUM_FIELDS

_NC, _NS, _L = 2, 16, 16
_NW = _NC * _NS
_B_PER_W = B_TOTAL // _NW
_CHUNK = 1664
_N_CHUNKS = _B_PER_W // _CHUNK


def _body(x_hbm, table_hbm, out_hbm, idx_v, rows_v, sem):
    wid = lax.axis_index("s") * _NC + lax.axis_index("c")
    base = wid * _B_PER_W

    def chunk_body(ci, carry):
        off = base + ci * _CHUNK
        pltpu.sync_copy(x_hbm.at[pl.ds(off, _CHUNK)], idx_v)

        def vec_body(i, c2):
            lane = i * _L + lax.iota(jnp.int32, _L)
            field = lax.rem(lane, NUM_FIELDS)
            idx_v[pl.ds(i * _L, _L)] = (
                idx_v[pl.ds(i * _L, _L)] + field * FIELD_SIZE
            )
            return c2

        lax.fori_loop(0, _CHUNK // _L, vec_body, 0, unroll=False)
        pltpu.async_copy(table_hbm.at[idx_v], rows_v, sem).wait()
        pltpu.sync_copy(rows_v, out_hbm.at[pl.ds(off, _CHUNK)])
        return carry

    lax.fori_loop(0, _N_CHUNKS, chunk_body, 0, unroll=False)


@functools.partial(jax.jit, static_argnums=())
def kernel(x, table):
    x_flat = x.reshape(B_TOTAL)
    mesh = plsc.VectorSubcoreMesh(core_axis_name="c", subcore_axis_name="s")
    run = functools.partial(
        pl.kernel,
        mesh=mesh,
        out_type=jax.ShapeDtypeStruct((B_TOTAL, DIM), jnp.float32),
        scratch_types=[
            pltpu.VMEM((_CHUNK,), jnp.int32),
            pltpu.VMEM((_CHUNK, DIM), jnp.float32),
            pltpu.SemaphoreType.DMA,
        ],
        compiler_params=pltpu.CompilerParams(use_tc_tiling_on_sc=False),
    )(_body)
    out = run(x_flat, table)
    return out.reshape(ROWS, NUM_FIELDS, DIM)
